# baseline (speedup 1.0000x reference)
import functools

import jax
import jax.numpy as jnp
from jax import lax
from jax.experimental import pallas as pl
from jax.experimental.pallas import tpu as pltpu
from jax.experimental.pallas import tpu_sc as plsc

N_NODES = 100000
D_FEAT = 128
NUM_GRAPHS = 512
D_OUT = 128

_NC = 2
_NS = 16
_NW = _NC * _NS
_SUB = 80
_NSUB = 2
_GC = _SUB * _NSUB
_NCHUNKS = N_NODES // _GC
_Q, _R = divmod(_NCHUNKS, _NW)
_QMAX = _Q + 1
_NCHUNKS_PAD = _QMAX * _NW
_LANES = 16
_GROWS = NUM_GRAPHS // _NS


def _make_seg_pool(cw):
  mesh = plsc.VectorSubcoreMesh(core_axis_name="c", subcore_axis_name="s")

  @functools.partial(
      pl.kernel,
      mesh=mesh,
      out_type=(
          jax.ShapeDtypeStruct((_NC, NUM_GRAPHS, D_FEAT), jnp.float32),
          jax.ShapeDtypeStruct((_NC, NUM_GRAPHS, cw), jnp.float32),
      ),
      scratch_types=(
          pltpu.VMEM((2, _GC, D_FEAT), jnp.float32),
          pltpu.VMEM((_QMAX, _NSUB, _SUB), jnp.int32),
          pltpu.VMEM((_GROWS, D_FEAT), jnp.float32),
          pltpu.VMEM((_SUB, cw), jnp.float32),
          pltpu.VMEM((_GROWS, cw), jnp.float32),
          pltpu.VMEM_SHARED((NUM_GRAPHS, D_FEAT), jnp.float32),
          pltpu.VMEM_SHARED((NUM_GRAPHS, cw), jnp.float32),
          pltpu.SemaphoreType.DMA,
          pltpu.SemaphoreType.DMA,
      ),
  )
  def k(nodes_hbm, idx_hbm, zrow_hbm, zcnt_hbm, ones_hbm, sums_hbm, cnts_hbm,
        rows_v, idx_v, zrow_v, ones_v, zcnt_v, sums_sh, cnts_sh, sem0, sem1):
    cid = lax.axis_index("c")
    sid = lax.axis_index("s")
    wid = sid * _NC + cid

    pltpu.sync_copy(zrow_hbm, zrow_v)
    pltpu.sync_copy(zcnt_hbm, zcnt_v)
    pltpu.sync_copy(ones_hbm, ones_v)

    pltpu.sync_copy(zrow_v, sums_sh.at[pl.ds(sid * _GROWS, _GROWS)])
    pltpu.sync_copy(zcnt_v, cnts_sh.at[pl.ds(sid * _GROWS, _GROWS)])
    plsc.subcore_barrier()

    start = wid * _Q + jnp.minimum(wid, _R)
    count = _Q + (wid < _R).astype(jnp.int32)

    pltpu.sync_copy(idx_hbm.at[pl.ds(start, _QMAX)], idx_v)

    sems = (sem0, sem1)

    def gather(kk, b):
      return pltpu.make_async_copy(
          nodes_hbm.at[pl.ds((start + kk) * _GC, _GC)], rows_v.at[b], sems[b])

    def issue(kk, b):
      pl.when(kk < count)(lambda: gather(kk, b).start())

    issue(0, 0)
    issue(1, 1)

    def body(g, _):
      for b in range(2):
        kk = 2 * g + b

        def work(kk=kk, b=b):
          gather(kk, b).wait()
          for j in range(_NSUB):
            pltpu.sync_copy(rows_v.at[b, pl.ds(j * _SUB, _SUB)],
                            sums_sh.at[idx_v.at[kk, j]], add=True)
            pltpu.sync_copy(ones_v, cnts_sh.at[idx_v.at[kk, j]], add=True)
          issue(kk + 2, b)
        pl.when(kk < count)(work)
      return 0
    lax.fori_loop(0, _QMAX // 2, body, 0)

    plsc.subcore_barrier()
    pltpu.sync_copy(sums_sh.at[pl.ds(sid * _GROWS, _GROWS)],
                    sums_hbm.at[cid, pl.ds(sid * _GROWS, _GROWS)])
    pltpu.sync_copy(cnts_sh.at[pl.ds(sid * _GROWS, _GROWS)],
                    cnts_hbm.at[cid, pl.ds(sid * _GROWS, _GROWS)])

  def run(nodes, idx3):
    zrow = jnp.zeros((_GROWS, D_FEAT), jnp.float32)
    zcnt = jnp.zeros((_GROWS, cw), jnp.float32)
    ones = jnp.ones((_SUB, cw), jnp.float32)
    return k(nodes, idx3, zrow, zcnt, ones)

  return run


_CW = 128
_seg_pool = _make_seg_pool(_CW)


def _head_body(ps_ref, pc_ref, w_ref, b_ref, o_ref):
  s = ps_ref[0] + ps_ref[1]
  c = pc_ref[0, :, 0:1] + pc_ref[1, :, 0:1]
  pooled = s / jnp.maximum(c, 1.0)
  o_ref[...] = lax.dot_general(
      pooled, w_ref[...], (((1,), (1,)), ((), ())),
      preferred_element_type=jnp.float32) + b_ref[...]


def kernel(node_representation, graph_index, W, b):
  idx3 = graph_index.astype(jnp.int32).reshape(_NCHUNKS, _NSUB, _SUB)
  idx3 = jnp.pad(idx3, ((0, _NCHUNKS_PAD - _NCHUNKS), (0, 0), (0, 0)))
  sums, cnts = _seg_pool(node_representation, idx3)
  out = pl.pallas_call(
      _head_body,
      out_shape=jax.ShapeDtypeStruct((NUM_GRAPHS, D_OUT), jnp.float32),
  )(sums, cnts, W, b.reshape(1, D_OUT))
  return out


# speedup vs baseline: 1.2323x; 1.2323x over previous
import functools

import jax
import jax.numpy as jnp
from jax import lax
from jax.experimental import pallas as pl
from jax.experimental.pallas import tpu as pltpu
from jax.experimental.pallas import tpu_sc as plsc

N_NODES = 100000
D_FEAT = 128
NUM_GRAPHS = 512
D_OUT = 128

_NC = 2
_NS = 16
_NW = _NC * _NS
_SUB = 80
_NSUB = 2
_GC = _SUB * _NSUB
_NCHUNKS = N_NODES // _GC
_Q, _R = divmod(_NCHUNKS, _NW)
_QMAX = _Q + 1
_NCHUNKS_PAD = _QMAX * _NW
_LANES = 16
_GROWS = NUM_GRAPHS // _NS
_NTMAX = _QMAX * _GC
_BCAP = NUM_GRAPHS + _LANES


def _seg_pool(nodes, idx3, idxflat, zrow, zcnt):
  mesh = plsc.VectorSubcoreMesh(core_axis_name="c", subcore_axis_name="s")

  @functools.partial(
      pl.kernel,
      mesh=mesh,
      out_type=(
          jax.ShapeDtypeStruct((_NC, NUM_GRAPHS, D_FEAT), jnp.float32),
          jax.ShapeDtypeStruct((_NC, _NS, NUM_GRAPHS), jnp.float32),
      ),
      scratch_types=(
          pltpu.VMEM((2, _GC, D_FEAT), jnp.float32),
          pltpu.VMEM((_QMAX, _NSUB, _SUB), jnp.int32),
          pltpu.VMEM((2 * _LANES + _NTMAX,), jnp.int32),
          pltpu.VMEM((_GROWS, D_FEAT), jnp.float32),
          pltpu.VMEM((NUM_GRAPHS,), jnp.float32),
          pltpu.VMEM_SHARED((NUM_GRAPHS, D_FEAT), jnp.float32),
          pltpu.SemaphoreType.DMA,
          pltpu.SemaphoreType.DMA,
      ),
  )
  def k(nodes_hbm, idx_hbm, idxflat_hbm, zrow_hbm, zcnt_hbm, sums_hbm, cnts_hbm,
        rows_v, idx_v, idxf_v, zrow_v, cnt_v, sums_sh, sem0, sem1):
    cid = lax.axis_index("c")
    sid = lax.axis_index("s")
    wid = sid * _NC + cid

    pltpu.sync_copy(zrow_hbm, zrow_v)
    pltpu.sync_copy(zcnt_hbm, cnt_v)

    pltpu.sync_copy(zrow_v, sums_sh.at[pl.ds(sid * _GROWS, _GROWS)])
    plsc.subcore_barrier()

    start = wid * _Q + jnp.minimum(wid, _R)
    count = _Q + (wid < _R).astype(jnp.int32)
    nt = count * _GC

    pltpu.sync_copy(idx_hbm.at[pl.ds(start, _QMAX)], idx_v)
    pltpu.sync_copy(idxflat_hbm.at[wid], idxf_v)

    sems = (sem0, sem1)

    def gather(kk, b):
      return pltpu.make_async_copy(
          nodes_hbm.at[pl.ds((start + kk) * _GC, _GC)], rows_v.at[b], sems[b])

    def issue(kk, b):
      pl.when(kk < count)(lambda: gather(kk, b).start())

    issue(0, 0)
    issue(1, 1)

    def body(g, _):
      for b in range(2):
        kk = 2 * g + b

        def work(kk=kk, b=b):
          gather(kk, b).wait()
          for j in range(_NSUB):
            pltpu.sync_copy(rows_v.at[b, pl.ds(j * _SUB, _SUB)],
                            sums_sh.at[idx_v.at[kk, j]], add=True)
          issue(kk + 2, b)
        pl.when(kk < count)(work)
      return 0
    lax.fori_loop(0, _QMAX // 2, body, 0)

    lanes = lax.iota(jnp.int32, _LANES)

    def scan_win(m, w):
      off = m * _LANES
      v0 = idxf_v[pl.ds(off + _LANES, _LANES)]
      pv = idxf_v[pl.ds(off + _LANES - 1, _LANES)]
      return w + jnp.where(v0 != pv, 1, 0)

    wv = lax.fori_loop(0, nt // _LANES, scan_win,
                       jnp.zeros((_LANES,), jnp.int32))
    nruns = sum(wv[kq] for kq in range(_LANES))

    def rdv(i):
      return idxf_v[pl.ds(i + _LANES, _LANES)][0]

    def run_body(r, i):
      g = rdv(i)

      def bs_body(_, s):
        lo, hi = s
        mid = (lo + hi) // 2
        adv = (lo < hi) & (rdv(mid) == g)
        return (jnp.where(adv, mid + 1, lo),
                jnp.where(lo < hi, jnp.where(adv, hi, mid), hi))

      e, _2 = lax.fori_loop(0, 12, bs_body, (i + 1, nt))
      gb = jnp.minimum(g, NUM_GRAPHS - _LANES)
      upd = jnp.where(lanes == (g - gb), (e - i).astype(jnp.float32), 0.0)
      cnt_v[pl.ds(gb, _LANES)] = cnt_v[pl.ds(gb, _LANES)] + upd
      return e

    lax.fori_loop(0, nruns, run_body, 0)

    pltpu.sync_copy(cnt_v, cnts_hbm.at[cid, sid])
    plsc.subcore_barrier()
    pltpu.sync_copy(sums_sh.at[pl.ds(sid * _GROWS, _GROWS)],
                    sums_hbm.at[cid, pl.ds(sid * _GROWS, _GROWS)])

  return k(nodes, idx3, idxflat, zrow, zcnt)


def _head_body(ps_ref, pc_ref, w_ref, b_ref, o_ref):
  s = ps_ref[0] + ps_ref[1]
  c = jnp.sum(pc_ref[...], axis=(0, 1)).reshape(NUM_GRAPHS, 1)
  pooled = s / jnp.maximum(c, 1.0)
  o_ref[...] = lax.dot_general(
      pooled, w_ref[...], (((1,), (1,)), ((), ())),
      preferred_element_type=jnp.float32) + b_ref[...]


def kernel(node_representation, graph_index, W, b):
  gi = graph_index.astype(jnp.int32)
  idx3 = gi.reshape(_NCHUNKS, _NSUB, _SUB)
  idx3 = jnp.pad(idx3, ((0, _NCHUNKS_PAD - _NCHUNKS), (0, 0), (0, 0)))
  gpad = jnp.pad(gi, (0, _NCHUNKS_PAD * _GC - N_NODES + _LANES))
  sent = jnp.full((_NW, _LANES), -1, jnp.int32)
  slices = [gpad[(w * _Q + min(w, _R)) * _GC:
                 (w * _Q + min(w, _R)) * _GC + _NTMAX + _LANES]
            for w in range(_NW)]
  idxflat = jnp.concatenate([sent, jnp.stack(slices)], axis=1)
  zrow = jnp.zeros((_GROWS, D_FEAT), jnp.float32)
  zcnt = jnp.zeros((NUM_GRAPHS,), jnp.float32)
  sums, cnts = _seg_pool(node_representation, idx3, idxflat, zrow, zcnt)
  out = pl.pallas_call(
      _head_body,
      out_shape=jax.ShapeDtypeStruct((NUM_GRAPHS, D_OUT), jnp.float32),
  )(sums, cnts, W, b.reshape(1, D_OUT))
  return out


# speedup vs baseline: 1.2711x; 1.0314x over previous
import functools

import jax
import jax.numpy as jnp
from jax import lax
from jax.experimental import pallas as pl
from jax.experimental.pallas import tpu as pltpu
from jax.experimental.pallas import tpu_sc as plsc

N_NODES = 100000
D_FEAT = 128
NUM_GRAPHS = 512
D_OUT = 128

_NC = 2
_NS = 16
_NW = _NC * _NS
_SUB = 80
_NSUB = 5
_GC = _SUB * _NSUB
_NCHUNKS = N_NODES // _GC
_Q, _R = divmod(_NCHUNKS, _NW)
_QMAX = _Q + 1
_NCHUNKS_PAD = _QMAX * _NW
_LANES = 16
_GROWS = NUM_GRAPHS // _NS
_NTMAX = _QMAX * _GC
_BCAP = NUM_GRAPHS + _LANES


def _seg_pool(nodes, idx3, idxflat, zrow, zcnt):
  mesh = plsc.VectorSubcoreMesh(core_axis_name="c", subcore_axis_name="s")

  @functools.partial(
      pl.kernel,
      mesh=mesh,
      out_type=(
          jax.ShapeDtypeStruct((_NC, NUM_GRAPHS, D_FEAT), jnp.float32),
          jax.ShapeDtypeStruct((_NC, _NS, NUM_GRAPHS), jnp.float32),
      ),
      scratch_types=(
          pltpu.VMEM((2, _GC, D_FEAT), jnp.float32),
          pltpu.VMEM((_QMAX, _NSUB, _SUB), jnp.int32),
          pltpu.VMEM((2 * _LANES + _NTMAX,), jnp.int32),
          pltpu.VMEM((_GROWS, D_FEAT), jnp.float32),
          pltpu.VMEM((NUM_GRAPHS,), jnp.float32),
          pltpu.VMEM_SHARED((NUM_GRAPHS, D_FEAT), jnp.float32),
          pltpu.SemaphoreType.DMA,
          pltpu.SemaphoreType.DMA,
          pltpu.SemaphoreType.DMA,
      ),
  )
  def k(nodes_hbm, idx_hbm, idxflat_hbm, zrow_hbm, zcnt_hbm, sums_hbm, cnts_hbm,
        rows_v, idx_v, idxf_v, zrow_v, cnt_v, sums_sh, sem0, sem1, ssem):
    cid = lax.axis_index("c")
    sid = lax.axis_index("s")
    wid = sid * _NC + cid

    pltpu.sync_copy(zrow_hbm, zrow_v)
    pltpu.sync_copy(zcnt_hbm, cnt_v)

    pltpu.sync_copy(zrow_v, sums_sh.at[pl.ds(sid * _GROWS, _GROWS)])
    plsc.subcore_barrier()

    start = wid * _Q + jnp.minimum(wid, _R)
    count = _Q + (wid < _R).astype(jnp.int32)
    nt = count * _GC

    pltpu.sync_copy(idx_hbm.at[pl.ds(start, _QMAX)], idx_v)
    pltpu.sync_copy(idxflat_hbm.at[wid], idxf_v)

    sems = (sem0, sem1)

    def gather(kk, b):
      return pltpu.make_async_copy(
          nodes_hbm.at[pl.ds((start + kk) * _GC, _GC)], rows_v.at[b], sems[b])

    def issue(kk, b):
      pl.when(kk < count)(lambda: gather(kk, b).start())

    issue(0, 0)
    issue(1, 1)

    def body(g, _):
      for b in range(2):
        kk = 2 * g + b

        def work(kk=kk, b=b):
          gather(kk, b).wait()
          descs = [
              pltpu.async_copy(rows_v.at[b, pl.ds(j * _SUB, _SUB)],
                               sums_sh.at[idx_v.at[kk, j]], ssem, add=True)
              for j in range(_NSUB)]
          for d in descs:
            d.wait()
          issue(kk + 2, b)
        pl.when(kk < count)(work)
      return 0
    lax.fori_loop(0, _QMAX // 2, body, 0)

    lanes = lax.iota(jnp.int32, _LANES)

    def scan_win(m, w):
      off = m * _LANES
      v0 = idxf_v[pl.ds(off + _LANES, _LANES)]
      pv = idxf_v[pl.ds(off + _LANES - 1, _LANES)]
      return w + jnp.where(v0 != pv, 1, 0)

    wv = lax.fori_loop(0, nt // _LANES, scan_win,
                       jnp.zeros((_LANES,), jnp.int32))
    nruns = sum(wv[kq] for kq in range(_LANES))

    def rdv(i):
      return idxf_v[pl.ds(i + _LANES, _LANES)][0]

    def run_body(r, i):
      g = rdv(i)

      def bs_body(_, s):
        lo, hi = s
        mid = (lo + hi) // 2
        adv = (lo < hi) & (rdv(mid) == g)
        return (jnp.where(adv, mid + 1, lo),
                jnp.where(lo < hi, jnp.where(adv, hi, mid), hi))

      e, _2 = lax.fori_loop(0, 12, bs_body, (i + 1, nt))
      gb = jnp.minimum(g, NUM_GRAPHS - _LANES)
      upd = jnp.where(lanes == (g - gb), (e - i).astype(jnp.float32), 0.0)
      cnt_v[pl.ds(gb, _LANES)] = cnt_v[pl.ds(gb, _LANES)] + upd
      return e

    lax.fori_loop(0, nruns, run_body, 0)

    pltpu.sync_copy(cnt_v, cnts_hbm.at[cid, sid])
    plsc.subcore_barrier()
    pltpu.sync_copy(sums_sh.at[pl.ds(sid * _GROWS, _GROWS)],
                    sums_hbm.at[cid, pl.ds(sid * _GROWS, _GROWS)])

  return k(nodes, idx3, idxflat, zrow, zcnt)


def _head_body(ps_ref, pc_ref, w_ref, b_ref, o_ref):
  s = ps_ref[0] + ps_ref[1]
  c = jnp.sum(pc_ref[...], axis=(0, 1)).reshape(NUM_GRAPHS, 1)
  pooled = s / jnp.maximum(c, 1.0)
  o_ref[...] = lax.dot_general(
      pooled, w_ref[...], (((1,), (1,)), ((), ())),
      preferred_element_type=jnp.float32) + b_ref[...]


def kernel(node_representation, graph_index, W, b):
  gi = graph_index.astype(jnp.int32)
  idx3 = gi.reshape(_NCHUNKS, _NSUB, _SUB)
  idx3 = jnp.pad(idx3, ((0, _NCHUNKS_PAD - _NCHUNKS), (0, 0), (0, 0)))
  gpad = jnp.pad(gi, (0, _NCHUNKS_PAD * _GC - N_NODES + _LANES))
  sent = jnp.full((_NW, _LANES), -1, jnp.int32)
  slices = [gpad[(w * _Q + min(w, _R)) * _GC:
                 (w * _Q + min(w, _R)) * _GC + _NTMAX + _LANES]
            for w in range(_NW)]
  idxflat = jnp.concatenate([sent, jnp.stack(slices)], axis=1)
  zrow = jnp.zeros((_GROWS, D_FEAT), jnp.float32)
  zcnt = jnp.zeros((NUM_GRAPHS,), jnp.float32)
  sums, cnts = _seg_pool(node_representation, idx3, idxflat, zrow, zcnt)
  out = pl.pallas_call(
      _head_body,
      out_shape=jax.ShapeDtypeStruct((NUM_GRAPHS, D_OUT), jnp.float32),
  )(sums, cnts, W, b.reshape(1, D_OUT))
  return out


# speedup vs baseline: 1.2724x; 1.0010x over previous
import functools

import jax
import jax.numpy as jnp
from jax import lax
from jax.experimental import pallas as pl
from jax.experimental.pallas import tpu as pltpu
from jax.experimental.pallas import tpu_sc as plsc

N_NODES = 100000
D_FEAT = 128
NUM_GRAPHS = 512
D_OUT = 128

_NC = 2
_NS = 16
_NW = _NC * _NS
_SUB = 80
_NSUB = 2
_NBUF = 5
_PF = 2
_GC = _SUB * _NSUB
_NCHUNKS = N_NODES // _GC
_Q, _R = divmod(_NCHUNKS, _NW)
_QMAX = _Q + 1
_NCHUNKS_PAD = _QMAX * _NW
_LANES = 16
_GROWS = NUM_GRAPHS // _NS
_NTMAX = _QMAX * _GC
_BCAP = NUM_GRAPHS + _LANES


def _seg_pool(nodes, idx3, idxflat, zrow, zcnt):
  mesh = plsc.VectorSubcoreMesh(core_axis_name="c", subcore_axis_name="s")

  @functools.partial(
      pl.kernel,
      mesh=mesh,
      out_type=(
          jax.ShapeDtypeStruct((_NC, NUM_GRAPHS, D_FEAT), jnp.float32),
          jax.ShapeDtypeStruct((_NC, _NS, NUM_GRAPHS), jnp.float32),
      ),
      scratch_types=(
          pltpu.VMEM((_NBUF, _GC, D_FEAT), jnp.float32),
          pltpu.VMEM((_QMAX, _NSUB, _SUB), jnp.int32),
          pltpu.VMEM((2 * _LANES + _NTMAX,), jnp.int32),
          pltpu.VMEM((_GROWS, D_FEAT), jnp.float32),
          pltpu.VMEM((NUM_GRAPHS,), jnp.float32),
          pltpu.VMEM_SHARED((NUM_GRAPHS, D_FEAT), jnp.float32),
      ) + (pltpu.SemaphoreType.DMA,) * (2 * _NBUF),
  )
  def k(nodes_hbm, idx_hbm, idxflat_hbm, zrow_hbm, zcnt_hbm, sums_hbm, cnts_hbm,
        rows_v, idx_v, idxf_v, zrow_v, cnt_v, sums_sh, *sems):
    cid = lax.axis_index("c")
    sid = lax.axis_index("s")
    wid = sid * _NC + cid

    pltpu.sync_copy(zrow_hbm, zrow_v)
    pltpu.sync_copy(zcnt_hbm, cnt_v)

    pltpu.sync_copy(zrow_v, sums_sh.at[pl.ds(sid * _GROWS, _GROWS)])
    plsc.subcore_barrier()

    start = wid * _Q + jnp.minimum(wid, _R)
    count = _Q + (wid < _R).astype(jnp.int32)
    nt = count * _GC

    pltpu.sync_copy(idx_hbm.at[pl.ds(start, _QMAX)], idx_v)
    pltpu.sync_copy(idxflat_hbm.at[wid], idxf_v)

    gsems = sems[:_NBUF]
    ssems = sems[_NBUF:]
    _DL = _NBUF - _PF

    def gather(kk, b):
      return pltpu.make_async_copy(
          nodes_hbm.at[pl.ds((start + kk) * _GC, _GC)], rows_v.at[b], gsems[b])

    def sdesc(kk, b, j):
      return pltpu.make_async_copy(rows_v.at[b, pl.ds(j * _SUB, _SUB)],
                                   sums_sh.at[idx_v.at[kk, j]], ssems[b])

    for p in range(_PF):
      pl.when(p < count)(lambda p=p: gather(p, p).start())

    def body(g, _):
      for b in range(_NBUF):
        kk = _NBUF * g + b
        bp = (b + _PF) % _NBUF

        def work(kk=kk, b=b):
          gather(kk, b).wait()
          for j in range(_NSUB):
            pltpu.async_copy(rows_v.at[b, pl.ds(j * _SUB, _SUB)],
                             sums_sh.at[idx_v.at[kk, j]], ssems[b], add=True)
        pl.when(kk < count)(work)

        def drain_prev(kk=kk, bp=bp):
          for j in range(_NSUB):
            sdesc(kk - _DL, bp, j).wait()
        pl.when((kk >= _DL) & (kk - _DL < count))(drain_prev)

        def prefetch(kk=kk, bp=bp):
          gather(kk + _PF, bp).start()
        pl.when(kk + _PF < count)(prefetch)
      return 0
    _NSLOTS = -(-_QMAX // _NBUF) * _NBUF
    lax.fori_loop(0, _NSLOTS // _NBUF, body, 0)

    for s in range(_NSLOTS - _DL, _NSLOTS):
      def drain_tail(s=s):
        for j in range(_NSUB):
          sdesc(s, s % _NBUF, j).wait()
      pl.when(s < count)(drain_tail)

    lanes = lax.iota(jnp.int32, _LANES)

    def scan_win(m, w):
      off = m * _LANES
      v0 = idxf_v[pl.ds(off + _LANES, _LANES)]
      pv = idxf_v[pl.ds(off + _LANES - 1, _LANES)]
      return w + jnp.where(v0 != pv, 1, 0)

    wv = lax.fori_loop(0, nt // _LANES, scan_win,
                       jnp.zeros((_LANES,), jnp.int32))
    nruns = sum(wv[kq] for kq in range(_LANES))

    def rdv(i):
      return idxf_v[pl.ds(i + _LANES, _LANES)][0]

    def run_body(r, i):
      g = rdv(i)

      def bs_body(_, s):
        lo, hi = s
        mid = (lo + hi) // 2
        adv = (lo < hi) & (rdv(mid) == g)
        return (jnp.where(adv, mid + 1, lo),
                jnp.where(lo < hi, jnp.where(adv, hi, mid), hi))

      e, _2 = lax.fori_loop(0, 12, bs_body, (i + 1, nt))
      gb = jnp.minimum(g, NUM_GRAPHS - _LANES)
      upd = jnp.where(lanes == (g - gb), (e - i).astype(jnp.float32), 0.0)
      cnt_v[pl.ds(gb, _LANES)] = cnt_v[pl.ds(gb, _LANES)] + upd
      return e

    lax.fori_loop(0, nruns, run_body, 0)

    pltpu.sync_copy(cnt_v, cnts_hbm.at[cid, sid])
    plsc.subcore_barrier()
    pltpu.sync_copy(sums_sh.at[pl.ds(sid * _GROWS, _GROWS)],
                    sums_hbm.at[cid, pl.ds(sid * _GROWS, _GROWS)])

  return k(nodes, idx3, idxflat, zrow, zcnt)


def _head_body(ps_ref, pc_ref, w_ref, b_ref, o_ref):
  s = ps_ref[0] + ps_ref[1]
  c = jnp.sum(pc_ref[...], axis=(0, 1)).reshape(NUM_GRAPHS, 1)
  pooled = s / jnp.maximum(c, 1.0)
  o_ref[...] = lax.dot_general(
      pooled, w_ref[...], (((1,), (1,)), ((), ())),
      preferred_element_type=jnp.float32) + b_ref[...]


def kernel(node_representation, graph_index, W, b):
  gi = graph_index.astype(jnp.int32)
  idx3 = gi.reshape(_NCHUNKS, _NSUB, _SUB)
  idx3 = jnp.pad(idx3, ((0, _NCHUNKS_PAD - _NCHUNKS), (0, 0), (0, 0)))
  gpad = jnp.pad(gi, (0, _NCHUNKS_PAD * _GC - N_NODES + _LANES))
  sent = jnp.full((_NW, _LANES), -1, jnp.int32)
  slices = [gpad[(w * _Q + min(w, _R)) * _GC:
                 (w * _Q + min(w, _R)) * _GC + _NTMAX + _LANES]
            for w in range(_NW)]
  idxflat = jnp.concatenate([sent, jnp.stack(slices)], axis=1)
  zrow = jnp.zeros((_GROWS, D_FEAT), jnp.float32)
  zcnt = jnp.zeros((NUM_GRAPHS,), jnp.float32)
  sums, cnts = _seg_pool(node_representation, idx3, idxflat, zrow, zcnt)
  out = pl.pallas_call(
      _head_body,
      out_shape=jax.ShapeDtypeStruct((NUM_GRAPHS, D_OUT), jnp.float32),
  )(sums, cnts, W, b.reshape(1, D_OUT))
  return out


# speedup vs baseline: 1.3002x; 1.0219x over previous
import functools

import jax
import jax.numpy as jnp
from jax import lax
from jax.experimental import pallas as pl
from jax.experimental.pallas import tpu as pltpu
from jax.experimental.pallas import tpu_sc as plsc

N_NODES = 100000
D_FEAT = 128
NUM_GRAPHS = 512
D_OUT = 128

_NC = 2
_NS = 16
_NW = _NC * _NS
_SUB = 80
_NSUB = 2
_NBUF = 5
_PF = 2
_GC = _SUB * _NSUB
_NCHUNKS = N_NODES // _GC
_Q, _R = divmod(_NCHUNKS, _NW)
_QMAX = _Q + 1
_NCHUNKS_PAD = _QMAX * _NW
_LANES = 16
_GROWS = NUM_GRAPHS // _NS
_NTMAX = _QMAX * _GC
_BCAP = NUM_GRAPHS + _LANES


def _seg_pool(nodes, idx3, idxflat, zrow, zcnt):
  mesh = plsc.VectorSubcoreMesh(core_axis_name="c", subcore_axis_name="s")

  @functools.partial(
      pl.kernel,
      mesh=mesh,
      out_type=(
          jax.ShapeDtypeStruct((_NC, NUM_GRAPHS, D_FEAT), jnp.float32),
          jax.ShapeDtypeStruct((_NC, _NS, NUM_GRAPHS), jnp.float32),
      ),
      scratch_types=(
          pltpu.VMEM((_NBUF, _GC, D_FEAT), jnp.float32),
          pltpu.VMEM((_QMAX, _NSUB, _SUB), jnp.int32),
          pltpu.VMEM((2 * _LANES + _NTMAX,), jnp.int32),
          pltpu.VMEM((_GROWS, D_FEAT), jnp.float32),
          pltpu.VMEM((NUM_GRAPHS,), jnp.float32),
          pltpu.VMEM_SHARED((NUM_GRAPHS, D_FEAT), jnp.float32),
      ) + (pltpu.SemaphoreType.DMA,) * (2 * _NBUF),
  )
  def k(nodes_hbm, idx_hbm, idxflat_hbm, zrow_hbm, zcnt_hbm, sums_hbm, cnts_hbm,
        rows_v, idx_v, idxf_v, zrow_v, cnt_v, sums_sh, *sems):
    cid = lax.axis_index("c")
    sid = lax.axis_index("s")
    wid = sid * _NC + cid

    pltpu.sync_copy(zrow_hbm, zrow_v)
    pltpu.sync_copy(zcnt_hbm, cnt_v)

    pltpu.sync_copy(zrow_v, sums_sh.at[pl.ds(sid * _GROWS, _GROWS)])
    plsc.subcore_barrier()

    start = wid * _Q + jnp.minimum(wid, _R)
    count = _Q + (wid < _R).astype(jnp.int32)
    nt = count * _GC

    pltpu.sync_copy(idx_hbm.at[pl.ds(start, _QMAX)], idx_v)
    pltpu.sync_copy(idxflat_hbm.at[pl.ds(start * _GC, _NTMAX)],
                    idxf_v.at[pl.ds(_LANES, _NTMAX)])

    gsems = sems[:_NBUF]
    ssems = sems[_NBUF:]
    _DL = _NBUF - _PF

    def gather(kk, b):
      return pltpu.make_async_copy(
          nodes_hbm.at[pl.ds((start + kk) * _GC, _GC)], rows_v.at[b], gsems[b])

    def sdesc(kk, b, j):
      return pltpu.make_async_copy(rows_v.at[b, pl.ds(j * _SUB, _SUB)],
                                   sums_sh.at[idx_v.at[kk, j]], ssems[b])

    for p in range(_PF):
      pl.when(p < count)(lambda p=p: gather(p, p).start())

    def body(g, _):
      for b in range(_NBUF):
        kk = _NBUF * g + b
        bp = (b + _PF) % _NBUF

        def work(kk=kk, b=b):
          gather(kk, b).wait()
          for j in range(_NSUB):
            pltpu.async_copy(rows_v.at[b, pl.ds(j * _SUB, _SUB)],
                             sums_sh.at[idx_v.at[kk, j]], ssems[b], add=True)
        pl.when(kk < count)(work)

        def drain_prev(kk=kk, bp=bp):
          for j in range(_NSUB):
            sdesc(kk - _DL, bp, j).wait()
        pl.when((kk >= _DL) & (kk - _DL < count))(drain_prev)

        def prefetch(kk=kk, bp=bp):
          gather(kk + _PF, bp).start()
        pl.when(kk + _PF < count)(prefetch)
      return 0
    _NSLOTS = -(-_QMAX // _NBUF) * _NBUF
    lax.fori_loop(0, _NSLOTS // _NBUF, body, 0)

    for s in range(_NSLOTS - _DL, _NSLOTS):
      def drain_tail(s=s):
        for j in range(_NSUB):
          sdesc(s, s % _NBUF, j).wait()
      pl.when(s < count)(drain_tail)

    lanes = lax.iota(jnp.int32, _LANES)

    def scan_win(m, w):
      off = m * _LANES
      v0 = idxf_v[pl.ds(off + _LANES, _LANES)]
      pv = idxf_v[pl.ds(off + _LANES - 1, _LANES)]
      return w + jnp.where(v0 != pv, 1, 0)

    wv = lax.fori_loop(0, nt // _LANES, scan_win,
                       jnp.zeros((_LANES,), jnp.int32))
    fixv = jnp.where(idxf_v[pl.ds(0, _LANES)][_LANES - 1]
                     == idxf_v[pl.ds(_LANES, _LANES)][0], 1, 0)
    nruns = sum(wv[kq] for kq in range(_LANES)) + fixv

    def rdv(i):
      return idxf_v[pl.ds(i + _LANES, _LANES)][0]

    def run_body(r, i):
      g = rdv(i)

      def bs_body(_, s):
        lo, hi = s
        mid = (lo + hi) // 2
        adv = (lo < hi) & (rdv(mid) == g)
        return (jnp.where(adv, mid + 1, lo),
                jnp.where(lo < hi, jnp.where(adv, hi, mid), hi))

      e, _2 = lax.fori_loop(0, 12, bs_body, (i + 1, nt))
      gb = jnp.minimum(g, NUM_GRAPHS - _LANES)
      upd = jnp.where(lanes == (g - gb), (e - i).astype(jnp.float32), 0.0)
      cnt_v[pl.ds(gb, _LANES)] = cnt_v[pl.ds(gb, _LANES)] + upd
      return e

    lax.fori_loop(0, nruns, run_body, 0)

    pltpu.sync_copy(cnt_v, cnts_hbm.at[cid, sid])
    plsc.subcore_barrier()
    pltpu.sync_copy(sums_sh.at[pl.ds(sid * _GROWS, _GROWS)],
                    sums_hbm.at[cid, pl.ds(sid * _GROWS, _GROWS)])

  return k(nodes, idx3, idxflat, zrow, zcnt)


def _head_body(ps_ref, pc_ref, w_ref, b_ref, o_ref):
  s = ps_ref[0] + ps_ref[1]
  c = jnp.sum(pc_ref[...], axis=(0, 1)).reshape(NUM_GRAPHS, 1)
  pooled = s / jnp.maximum(c, 1.0)
  o_ref[...] = lax.dot_general(
      pooled, w_ref[...], (((1,), (1,)), ((), ())),
      preferred_element_type=jnp.float32) + b_ref[...]


def kernel(node_representation, graph_index, W, b):
  gi = graph_index.astype(jnp.int32)
  idx3 = gi.reshape(_NCHUNKS, _NSUB, _SUB)
  idx3 = jnp.pad(idx3, ((0, _NCHUNKS_PAD - _NCHUNKS), (0, 0), (0, 0)))
  idxflat = jnp.pad(gi, (0, _NCHUNKS_PAD * _GC - N_NODES))
  zrow = jnp.zeros((_GROWS, D_FEAT), jnp.float32)
  zcnt = jnp.zeros((NUM_GRAPHS,), jnp.float32)
  sums, cnts = _seg_pool(node_representation, idx3, idxflat, zrow, zcnt)
  out = pl.pallas_call(
      _head_body,
      out_shape=jax.ShapeDtypeStruct((NUM_GRAPHS, D_OUT), jnp.float32),
  )(sums, cnts, W, b.reshape(1, D_OUT))
  return out


# speedup vs baseline: 1.3436x; 1.0334x over previous
import functools

import jax
import jax.numpy as jnp
from jax import lax
from jax.experimental import pallas as pl
from jax.experimental.pallas import tpu as pltpu
from jax.experimental.pallas import tpu_sc as plsc

N_NODES = 100000
D_FEAT = 128
NUM_GRAPHS = 512
D_OUT = 128

_NC = 2
_NS = 16
_NW = _NC * _NS
_SUB = 80
_NSUB = 2
_NBUF = 5
_PF = 2
_GC = _SUB * _NSUB
_NCHUNKS = N_NODES // _GC
_Q, _R = divmod(_NCHUNKS, _NW)
_QMAX = _Q + 1
_NCHUNKS_PAD = _QMAX * _NW
_LANES = 16
_GROWS = NUM_GRAPHS // _NS
_NTMAX = _QMAX * _GC
_BCAP = NUM_GRAPHS + _LANES


def _seg_pool(nodes, idx3, idxflat, zrow, zcnt):
  mesh = plsc.VectorSubcoreMesh(core_axis_name="c", subcore_axis_name="s")

  @functools.partial(
      pl.kernel,
      mesh=mesh,
      out_type=(
          jax.ShapeDtypeStruct((_NC, NUM_GRAPHS, D_FEAT), jnp.float32),
          jax.ShapeDtypeStruct((_NC, _NS, NUM_GRAPHS), jnp.float32),
      ),
      scratch_types=(
          pltpu.VMEM((_NBUF, _GC, D_FEAT), jnp.float32),
          pltpu.VMEM((_QMAX, _NSUB, _SUB), jnp.int32),
          pltpu.VMEM((2 * _LANES + _NTMAX,), jnp.int32),
          pltpu.VMEM((_GROWS, D_FEAT), jnp.float32),
          pltpu.VMEM((NUM_GRAPHS,), jnp.float32),
          pltpu.VMEM_SHARED((NUM_GRAPHS, D_FEAT), jnp.float32),
      ) + (pltpu.SemaphoreType.DMA,) * (2 * _NBUF),
  )
  def k(nodes_hbm, idx_hbm, idxflat_hbm, zrow_hbm, zcnt_hbm, sums_hbm, cnts_hbm,
        rows_v, idx_v, idxf_v, zrow_v, cnt_v, sums_sh, *sems):
    cid = lax.axis_index("c")
    sid = lax.axis_index("s")
    wid = sid * _NC + cid

    pltpu.sync_copy(zrow_hbm, zrow_v)
    pltpu.sync_copy(zcnt_hbm, cnt_v)

    pltpu.sync_copy(zrow_v, sums_sh.at[pl.ds(sid * _GROWS, _GROWS)])
    plsc.subcore_barrier()

    start = wid * _Q + jnp.minimum(wid, _R)
    count = _Q + (wid < _R).astype(jnp.int32)
    nt = count * _GC

    st2 = jnp.minimum(start, _NCHUNKS - _QMAX)
    d = start - st2
    dd = d * _GC
    pltpu.sync_copy(idx_hbm.at[pl.ds(st2, _QMAX)], idx_v)
    pltpu.sync_copy(idxflat_hbm.at[pl.ds(st2 * _GC, _NTMAX)],
                    idxf_v.at[pl.ds(_LANES, _NTMAX)])

    gsems = sems[:_NBUF]
    ssems = sems[_NBUF:]
    _DL = _NBUF - _PF

    def gather(kk, b):
      return pltpu.make_async_copy(
          nodes_hbm.at[pl.ds((start + kk) * _GC, _GC)], rows_v.at[b], gsems[b])

    def sdesc(kk, b, j):
      return pltpu.make_async_copy(rows_v.at[b, pl.ds(j * _SUB, _SUB)],
                                   sums_sh.at[idx_v.at[kk + d, j]], ssems[b])

    for p in range(_PF):
      pl.when(p < count)(lambda p=p: gather(p, p).start())

    def body(g, _):
      for b in range(_NBUF):
        kk = _NBUF * g + b
        bp = (b + _PF) % _NBUF

        def work(kk=kk, b=b):
          gather(kk, b).wait()
          for j in range(_NSUB):
            pltpu.async_copy(rows_v.at[b, pl.ds(j * _SUB, _SUB)],
                             sums_sh.at[idx_v.at[kk + d, j]], ssems[b], add=True)
        pl.when(kk < count)(work)

        def drain_prev(kk=kk, bp=bp):
          for j in range(_NSUB):
            sdesc(kk - _DL, bp, j).wait()
        pl.when((kk >= _DL) & (kk - _DL < count))(drain_prev)

        def prefetch(kk=kk, bp=bp):
          gather(kk + _PF, bp).start()
        pl.when(kk + _PF < count)(prefetch)
      return 0
    _NSLOTS = -(-_QMAX // _NBUF) * _NBUF
    lax.fori_loop(0, _NSLOTS // _NBUF, body, 0)

    for s in range(_NSLOTS - _DL, _NSLOTS):
      def drain_tail(s=s):
        for j in range(_NSUB):
          sdesc(s, s % _NBUF, j).wait()
      pl.when(s < count)(drain_tail)

    lanes = lax.iota(jnp.int32, _LANES)

    def scan_win(m, w):
      off = m * _LANES + dd
      v0 = idxf_v[pl.ds(off + _LANES, _LANES)]
      pv = idxf_v[pl.ds(off + _LANES - 1, _LANES)]
      return w + jnp.where(v0 != pv, 1, 0)

    wv = lax.fori_loop(0, nt // _LANES, scan_win,
                       jnp.zeros((_LANES,), jnp.int32))
    fixv = jnp.where(idxf_v[pl.ds(dd, _LANES)][_LANES - 1]
                     == idxf_v[pl.ds(dd + _LANES, _LANES)][0], 1, 0)
    nruns = sum(wv[kq] for kq in range(_LANES)) + fixv

    def rdv(i):
      return idxf_v[pl.ds(i + dd + _LANES, _LANES)][0]

    def run_body(r, i):
      g = rdv(i)

      def bs_body(_, s):
        lo, hi = s
        mid = (lo + hi) // 2
        adv = (lo < hi) & (rdv(mid) == g)
        return (jnp.where(adv, mid + 1, lo),
                jnp.where(lo < hi, jnp.where(adv, hi, mid), hi))

      e, _2 = lax.fori_loop(0, 12, bs_body, (i + 1, nt))
      gb = jnp.minimum(g, NUM_GRAPHS - _LANES)
      upd = jnp.where(lanes == (g - gb), (e - i).astype(jnp.float32), 0.0)
      cnt_v[pl.ds(gb, _LANES)] = cnt_v[pl.ds(gb, _LANES)] + upd
      return e

    lax.fori_loop(0, nruns, run_body, 0)

    pltpu.sync_copy(cnt_v, cnts_hbm.at[cid, sid])
    plsc.subcore_barrier()
    pltpu.sync_copy(sums_sh.at[pl.ds(sid * _GROWS, _GROWS)],
                    sums_hbm.at[cid, pl.ds(sid * _GROWS, _GROWS)])

  return k(nodes, idx3, idxflat, zrow, zcnt)


def _head_body(ps_ref, pc_ref, w_ref, b_ref, o_ref):
  s = ps_ref[0] + ps_ref[1]
  c = jnp.sum(pc_ref[...], axis=(0, 1)).reshape(NUM_GRAPHS, 1)
  pooled = s / jnp.maximum(c, 1.0)
  o_ref[...] = lax.dot_general(
      pooled, w_ref[...], (((1,), (1,)), ((), ())),
      preferred_element_type=jnp.float32) + b_ref[...]


def kernel(node_representation, graph_index, W, b):
  gi = graph_index.astype(jnp.int32)
  idx3 = gi.reshape(_NCHUNKS, _NSUB, _SUB)
  idxflat = gi
  zrow = jnp.zeros((_GROWS, D_FEAT), jnp.float32)
  zcnt = jnp.zeros((NUM_GRAPHS,), jnp.float32)
  sums, cnts = _seg_pool(node_representation, idx3, idxflat, zrow, zcnt)
  out = pl.pallas_call(
      _head_body,
      out_shape=jax.ShapeDtypeStruct((NUM_GRAPHS, D_OUT), jnp.float32),
  )(sums, cnts, W, b.reshape(1, D_OUT))
  return out
